# baseline (speedup 1.0000x reference)
import dataclasses
import functools

import jax
import jax.numpy as jnp
from jax import lax
from jax.experimental import pallas as pl
from jax.experimental.pallas import tpu as pltpu
from jax.experimental.pallas import tpu_sc as plsc

RES = 100000
NUM_MOVING = 1500000
NUM_STILL = 500000
NUM_TRAIN_SAMPLES = 10000

R_ROWS = 8
R_COLS = 12544
BINS_PAD = R_ROWS * R_COLS

NC = 2
NS = 16
NW = NC * NS
LANES = 16

LW = 128
BR = 56
PAD_ROWS = 16384
PRE_GRID = 32
PRE_BR = PAD_ROWS // PRE_GRID

W_MOV = 1.0 / float(NUM_MOVING)
W_STL = 1.0 / float(NUM_STILL)


def _pre_body(stat_ref, dyn_ref, mask_ref, score_ref, impr_ref, idx_ref, *,
              nrows):
    i = pl.program_id(0)
    rowg = lax.broadcasted_iota(jnp.int32, (PRE_BR, LW), 0) + i * PRE_BR
    valid = rowg < nrows
    diff = stat_ref[...] - dyn_ref[...]
    w = jnp.where(mask_ref[...], jnp.float32(W_MOV), jnp.float32(W_STL))
    impr_ref[...] = jnp.where(valid, diff * w, 0.0)
    idx = (score_ref[...] * 100000.0).astype(jnp.int32)
    idx_ref[...] = jnp.where(valid, jnp.minimum(idx, RES - 1), 0)


def _tc_pre(stat2, dyn2, mask2, score2):
    nrows = stat2.shape[0]
    nblk_in = (nrows + PRE_BR - 1) // PRE_BR
    in_spec = pl.BlockSpec((PRE_BR, LW),
                           lambda i: (jnp.minimum(i, nblk_in - 1), 0))
    out_spec = pl.BlockSpec((PRE_BR, LW), lambda i: (i, 0))
    return pl.pallas_call(
        functools.partial(_pre_body, nrows=nrows),
        grid=(PRE_GRID,),
        in_specs=[in_spec] * 4,
        out_specs=(out_spec, out_spec),
        out_shape=(jax.ShapeDtypeStruct((PAD_ROWS, LW), jnp.float32),
                   jax.ShapeDtypeStruct((PAD_ROWS, LW), jnp.int32)),
    )(stat2, dyn2, mask2, score2)


def _sc_hist_fn(nrows):
    cr = (nrows // (NW * 8)) * 8
    tail = nrows - NW * cr
    nblk_full = cr // BR
    part = cr - nblk_full * BR

    mesh = plsc.VectorSubcoreMesh(core_axis_name="c", subcore_axis_name="s")

    cp = pltpu.CompilerParams()
    if "needs_layout_passes" in pltpu.CompilerParams.__dataclass_fields__:
        cp = dataclasses.replace(cp, needs_layout_passes=False)

    @functools.partial(
        pl.kernel,
        compiler_params=cp,
        out_type=jax.ShapeDtypeStruct((NW, R_ROWS, R_COLS), jnp.float32),
        mesh=mesh,
        scratch_types=[
            pltpu.VMEM((BINS_PAD,), jnp.float32),
            pltpu.VMEM((BR, LW), jnp.float32),
            pltpu.VMEM((BR, LW), jnp.int32),
            pltpu.VMEM((BR, LW), jnp.float32),
            pltpu.VMEM((BR, LW), jnp.int32),
            pltpu.SemaphoreType.DMA,
            pltpu.SemaphoreType.DMA,
        ],
    )
    def sc_hist(impr_hbm, idx_hbm, out_hbm, hist, bv0, bi0, bv1, bi1,
                sem0, sem1):
        wid = lax.axis_index("c") * NS + lax.axis_index("s")
        sems = (sem0, sem1)
        bufsets = ((bv0, bi0), (bv1, bi1))

        def issue(slot, row0, nr):
            bv, bi = bufsets[slot]
            dv = bv if nr == BR else bv.at[pl.ds(0, nr)]
            di = bi if nr == BR else bi.at[pl.ds(0, nr)]
            return [pltpu.async_copy(impr_hbm.at[pl.ds(row0, nr)], dv, sems[slot]),
                    pltpu.async_copy(idx_hbm.at[pl.ds(row0, nr)], di, sems[slot])]

        def compute(slot, nr):
            bv, bi = bufsets[slot]

            @plsc.parallel_loop(0, nr, unroll=2)
            def _rows(i):
                for t in range(LW // LANES):
                    sl = pl.ds(t * LANES, LANES)
                    plsc.addupdate_scatter(hist, [bi[i, sl]], bv[i, sl])

        items = [(b * BR, BR) for b in range(nblk_full)]
        if part > 0:
            items.append((nblk_full * BR, part))

        base0 = wid * cr
        cps = issue(0, base0 + items[0][0], items[0][1])

        @plsc.parallel_loop(0, BINS_PAD, step=8 * LANES, unroll=2)
        def _zero(i):
            for t in range(8):
                hist[pl.ds(i + t * LANES, LANES)] = jnp.zeros((LANES,), jnp.float32)
        for j in range(len(items)):
            slot = j % 2
            nxt = j + 1
            if nxt < len(items):
                pre = issue(nxt % 2, base0 + items[nxt][0], items[nxt][1])
            for c in cps:
                c.wait()
            compute(slot, items[j][1])
            if nxt < len(items):
                cps = pre

        if tail > 0:
            @pl.when(wid == NW - 1)
            def _tail():
                tcp = issue(0, NW * cr, tail)
                for c in tcp:
                    c.wait()
                compute(0, tail)

        for r in range(R_ROWS):
            pltpu.sync_copy(hist.at[pl.ds(r * R_COLS, R_COLS)],
                            out_hbm.at[wid, r])

    return sc_hist


def _post_body(x_ref, o_ref, *, scale):
    h = jnp.sum(x_ref[...], axis=0)
    m = h * scale

    lane = lax.broadcasted_iota(jnp.int32, (R_ROWS, R_COLS), 1)
    row = lax.broadcasted_iota(jnp.int32, (R_ROWS, R_COLS), 0)

    c = m
    k = 1
    while k < R_COLS:
        shifted = pltpu.roll(c, k, 1)
        c = c + jnp.where(lane >= k, shifted, 0.0)
        k *= 2

    rs = c[:, R_COLS - 1:R_COLS]
    rowi = lax.broadcasted_iota(jnp.int32, (R_ROWS, 1), 0)
    rc = rs
    for k in (1, 2, 4):
        shifted = pltpu.roll(rc, k, 0)
        rc = rc + jnp.where(rowi >= k, shifted, 0.0)

    prefix = c + (rc - rs)

    flat = lane + R_COLS * row
    valid = flat < RES
    pmin = jnp.min(jnp.where(valid, prefix, jnp.inf))
    best = jnp.minimum(pmin, 0.0)

    is_best = jnp.logical_and(valid, prefix == best)
    cnt = jnp.sum(is_best.astype(jnp.float32)) + jnp.where(
        best == 0.0, jnp.float32(1.0), jnp.float32(0.0))
    sidx = jnp.sum(jnp.where(is_best, (flat + 1).astype(jnp.float32), 0.0))
    o_ref[0, 0] = sidx / cnt / 100000.0


def _tc_post(x, scale, interpret=False):
    return pl.pallas_call(
        functools.partial(_post_body, scale=scale),
        out_shape=jax.ShapeDtypeStruct((1, 1), jnp.float32),
        out_specs=pl.BlockSpec(memory_space=pltpu.SMEM),
        interpret=interpret,
    )(x)


def kernel(epes_stat_flow, epes_dyn_flow, moving_mask, dynamicness_scores,
           summaries=0, training=True):
    n = epes_stat_flow.shape[0]
    total = NUM_MOVING + NUM_STILL
    avg_points_per_sample = total / NUM_TRAIN_SAMPLES
    update_weight = 1.0 / min(2.0 * total, 5000.0 * avg_points_per_sample)
    cur_update_weight = (1.0 - update_weight) ** float(n)
    scale = 1.0 - cur_update_weight

    nrows = n // LW
    stat2 = epes_stat_flow.reshape(nrows, LW)
    dyn2 = epes_dyn_flow.reshape(nrows, LW)
    mask2 = moving_mask.reshape(nrows, LW)
    score2 = dynamicness_scores.reshape(nrows, LW)

    impr, idx = _tc_pre(stat2, dyn2, mask2, score2)
    x = _sc_hist_fn(PAD_ROWS)(impr, idx)
    out = _tc_post(x, scale)
    return out.reshape((1,))


# speedup vs baseline: 1.8515x; 1.8515x over previous
import dataclasses
import functools

import jax
import jax.numpy as jnp
from jax import lax
from jax.experimental import pallas as pl
from jax.experimental.pallas import tpu as pltpu
from jax.experimental.pallas import tpu_sc as plsc

RES = 100000
NUM_MOVING = 1500000
NUM_STILL = 500000
NUM_TRAIN_SAMPLES = 10000

R_ROWS = 8
R_COLS = 12544
BINS_PAD = R_ROWS * R_COLS

NC = 2
NS = 16
NW = NC * NS
LANES = 16

LW = 128
BR = 56
PRE_BR = 512
PRE_GRID = 31
PAD_ROWS = PRE_BR * PRE_GRID

W_MOV = 1.0 / float(NUM_MOVING)
W_STL = 1.0 / float(NUM_STILL)


def _pre_body(stat_ref, dyn_ref, mask_ref, score_ref, impr_ref, idx_ref, *,
              nrows):
    i = pl.program_id(0)
    rowg = lax.broadcasted_iota(jnp.int32, (PRE_BR, LW), 0) + i * PRE_BR
    valid = rowg < nrows
    diff = stat_ref[...] - dyn_ref[...]
    w = jnp.where(mask_ref[...], jnp.float32(W_MOV), jnp.float32(W_STL))
    impr_ref[...] = jnp.where(valid, diff * w, 0.0)
    idx = (score_ref[...] * 100000.0).astype(jnp.int32)
    lane = lax.broadcasted_iota(jnp.int32, (PRE_BR, LW), 1)
    idx_ref[...] = jnp.where(valid, jnp.minimum(idx, RES - 1), lane)


def _tc_pre(stat2, dyn2, mask2, score2):
    nrows = stat2.shape[0]
    in_spec = pl.BlockSpec((PRE_BR, LW), lambda i: (i, 0))
    out_spec = pl.BlockSpec((PRE_BR, LW), lambda i: (i, 0))
    return pl.pallas_call(
        functools.partial(_pre_body, nrows=nrows),
        grid=(PRE_GRID,),
        in_specs=[in_spec] * 4,
        out_specs=(out_spec, out_spec),
        out_shape=(jax.ShapeDtypeStruct((PAD_ROWS, LW), jnp.float32),
                   jax.ShapeDtypeStruct((PAD_ROWS, LW), jnp.int32)),
    )(stat2, dyn2, mask2, score2)


def _sc_hist_fn(nrows):
    cr = (nrows // (NW * 8)) * 8
    tail = nrows - NW * cr
    nblk_full = cr // BR
    part = cr - nblk_full * BR

    mesh = plsc.VectorSubcoreMesh(core_axis_name="c", subcore_axis_name="s")

    cp = pltpu.CompilerParams()
    if "needs_layout_passes" in pltpu.CompilerParams.__dataclass_fields__:
        cp = dataclasses.replace(cp, needs_layout_passes=False)

    @functools.partial(
        pl.kernel,
        compiler_params=cp,
        out_type=jax.ShapeDtypeStruct((NW, R_ROWS, R_COLS), jnp.float32),
        mesh=mesh,
        scratch_types=[
            pltpu.VMEM((BINS_PAD,), jnp.float32),
            pltpu.VMEM((BR, LW), jnp.float32),
            pltpu.VMEM((BR, LW), jnp.int32),
            pltpu.VMEM((BR, LW), jnp.float32),
            pltpu.VMEM((BR, LW), jnp.int32),
            pltpu.SemaphoreType.DMA,
            pltpu.SemaphoreType.DMA,
        ],
    )
    def sc_hist(impr_hbm, idx_hbm, out_hbm, hist, bv0, bi0, bv1, bi1,
                sem0, sem1):
        wid = lax.axis_index("c") * NS + lax.axis_index("s")
        sems = (sem0, sem1)
        bufsets = ((bv0, bi0), (bv1, bi1))

        def issue(slot, row0, nr):
            bv, bi = bufsets[slot]
            dv = bv if nr == BR else bv.at[pl.ds(0, nr)]
            di = bi if nr == BR else bi.at[pl.ds(0, nr)]
            return [pltpu.async_copy(impr_hbm.at[pl.ds(row0, nr)], dv, sems[slot]),
                    pltpu.async_copy(idx_hbm.at[pl.ds(row0, nr)], di, sems[slot])]

        def compute(slot, nr):
            bv, bi = bufsets[slot]

            @plsc.parallel_loop(0, nr, unroll=2)
            def _rows(i):
                for t in range(LW // LANES):
                    sl = pl.ds(t * LANES, LANES)
                    plsc.addupdate_scatter(hist, [bi[i, sl]], bv[i, sl])

        items = [(b * BR, BR) for b in range(nblk_full)]
        if part > 0:
            items.append((nblk_full * BR, part))

        base0 = wid * cr
        cps = issue(0, base0 + items[0][0], items[0][1])

        @plsc.parallel_loop(0, BINS_PAD, step=8 * LANES, unroll=2)
        def _zero(i):
            for t in range(8):
                hist[pl.ds(i + t * LANES, LANES)] = jnp.zeros((LANES,), jnp.float32)
        for j in range(len(items)):
            slot = j % 2
            nxt = j + 1
            if nxt < len(items):
                pre = issue(nxt % 2, base0 + items[nxt][0], items[nxt][1])
            for c in cps:
                c.wait()
            compute(slot, items[j][1])
            if nxt < len(items):
                cps = pre

        if tail > 0:
            @pl.when(wid == NW - 1)
            def _tail():
                tcp = issue(0, NW * cr, tail)
                for c in tcp:
                    c.wait()
                compute(0, tail)

        for r in range(R_ROWS):
            pltpu.sync_copy(hist.at[pl.ds(r * R_COLS, R_COLS)],
                            out_hbm.at[wid, r])

    return sc_hist


def _post_body(x_ref, o_ref, *, scale):
    h = jnp.sum(x_ref[...], axis=0)
    m = h * scale

    lane = lax.broadcasted_iota(jnp.int32, (R_ROWS, R_COLS), 1)
    row = lax.broadcasted_iota(jnp.int32, (R_ROWS, R_COLS), 0)

    c = m
    k = 1
    while k < R_COLS:
        shifted = pltpu.roll(c, k, 1)
        c = c + jnp.where(lane >= k, shifted, 0.0)
        k *= 2

    rs = c[:, R_COLS - 1:R_COLS]
    rowi = lax.broadcasted_iota(jnp.int32, (R_ROWS, 1), 0)
    rc = rs
    for k in (1, 2, 4):
        shifted = pltpu.roll(rc, k, 0)
        rc = rc + jnp.where(rowi >= k, shifted, 0.0)

    prefix = c + (rc - rs)

    flat = lane + R_COLS * row
    valid = flat < RES
    pmin = jnp.min(jnp.where(valid, prefix, jnp.inf))
    best = jnp.minimum(pmin, 0.0)

    is_best = jnp.logical_and(valid, prefix == best)
    cnt = jnp.sum(is_best.astype(jnp.float32)) + jnp.where(
        best == 0.0, jnp.float32(1.0), jnp.float32(0.0))
    sidx = jnp.sum(jnp.where(is_best, (flat + 1).astype(jnp.float32), 0.0))
    o_ref[0, 0] = sidx / cnt / 100000.0


def _tc_post(x, scale, interpret=False):
    return pl.pallas_call(
        functools.partial(_post_body, scale=scale),
        out_shape=jax.ShapeDtypeStruct((1, 1), jnp.float32),
        out_specs=pl.BlockSpec(memory_space=pltpu.SMEM),
        interpret=interpret,
    )(x)


def kernel(epes_stat_flow, epes_dyn_flow, moving_mask, dynamicness_scores,
           summaries=0, training=True):
    n = epes_stat_flow.shape[0]
    total = NUM_MOVING + NUM_STILL
    avg_points_per_sample = total / NUM_TRAIN_SAMPLES
    update_weight = 1.0 / min(2.0 * total, 5000.0 * avg_points_per_sample)
    cur_update_weight = (1.0 - update_weight) ** float(n)
    scale = 1.0 - cur_update_weight

    nrows = n // LW
    stat2 = epes_stat_flow.reshape(nrows, LW)
    dyn2 = epes_dyn_flow.reshape(nrows, LW)
    mask2 = moving_mask.reshape(nrows, LW)
    score2 = dynamicness_scores.reshape(nrows, LW)

    impr, idx = _tc_pre(stat2, dyn2, mask2, score2)
    x = _sc_hist_fn(PAD_ROWS)(impr, idx)
    out = _tc_post(x, scale)
    return out.reshape((1,))


# speedup vs baseline: 2.1448x; 1.1584x over previous
import dataclasses
import functools

import jax
import jax.numpy as jnp
from jax import lax
from jax.experimental import pallas as pl
from jax.experimental.pallas import tpu as pltpu
from jax.experimental.pallas import tpu_sc as plsc

RES = 100000
NUM_MOVING = 1500000
NUM_STILL = 500000
NUM_TRAIN_SAMPLES = 10000

R_ROWS = 8
R_COLS = 12544
BINS_PAD = R_ROWS * R_COLS

NC = 2
NS = 16
NW = NC * NS
LANES = 16

LW = 128
BR = 56
PRE_BR = 512
PRE_GRID = 31
PAD_ROWS = PRE_BR * PRE_GRID

W_MOV = 1.0 / float(NUM_MOVING)
W_STL = 1.0 / float(NUM_STILL)


def _pre_body(stat_ref, dyn_ref, mask_ref, score_ref, impr_ref, idx_ref, *,
              nrows):
    diff = stat_ref[...] - dyn_ref[...]
    w = jnp.where(mask_ref[...], jnp.float32(W_MOV), jnp.float32(W_STL))
    impr = diff * w
    idx = jnp.minimum((score_ref[...] * 100000.0).astype(jnp.int32), RES - 1)
    pad = PAD_ROWS - nrows
    if pad > 0:
        impr = jnp.concatenate(
            [impr, jnp.zeros((pad, LW), jnp.float32)], axis=0)
        idx = jnp.concatenate(
            [idx, lax.broadcasted_iota(jnp.int32, (pad, LW), 1)], axis=0)
    impr_ref[...] = impr
    idx_ref[...] = idx


def _tc_pre(stat2, dyn2, mask2, score2):
    nrows = stat2.shape[0]
    return pl.pallas_call(
        functools.partial(_pre_body, nrows=nrows),
        out_shape=(jax.ShapeDtypeStruct((PAD_ROWS, LW), jnp.float32),
                   jax.ShapeDtypeStruct((PAD_ROWS, LW), jnp.int32)),
    )(stat2, dyn2, mask2, score2)


def _sc_hist_fn(nrows):
    cr = (nrows // (NW * 8)) * 8
    tail = nrows - NW * cr
    nblk_full = cr // BR
    part = cr - nblk_full * BR

    mesh = plsc.VectorSubcoreMesh(core_axis_name="c", subcore_axis_name="s")

    cp = pltpu.CompilerParams()
    if "needs_layout_passes" in pltpu.CompilerParams.__dataclass_fields__:
        cp = dataclasses.replace(cp, needs_layout_passes=False)

    @functools.partial(
        pl.kernel,
        compiler_params=cp,
        out_type=jax.ShapeDtypeStruct((NW, R_ROWS, R_COLS), jnp.float32),
        mesh=mesh,
        scratch_types=[
            pltpu.VMEM((BINS_PAD,), jnp.float32),
            pltpu.VMEM((BR, LW), jnp.float32),
            pltpu.VMEM((BR, LW), jnp.int32),
            pltpu.VMEM((BR, LW), jnp.float32),
            pltpu.VMEM((BR, LW), jnp.int32),
            pltpu.SemaphoreType.DMA,
            pltpu.SemaphoreType.DMA,
        ],
    )
    def sc_hist(impr_hbm, idx_hbm, out_hbm, hist, bv0, bi0, bv1, bi1,
                sem0, sem1):
        wid = lax.axis_index("c") * NS + lax.axis_index("s")
        sems = (sem0, sem1)
        bufsets = ((bv0, bi0), (bv1, bi1))

        def issue(slot, row0, nr):
            bv, bi = bufsets[slot]
            dv = bv if nr == BR else bv.at[pl.ds(0, nr)]
            di = bi if nr == BR else bi.at[pl.ds(0, nr)]
            return [pltpu.async_copy(impr_hbm.at[pl.ds(row0, nr)], dv, sems[slot]),
                    pltpu.async_copy(idx_hbm.at[pl.ds(row0, nr)], di, sems[slot])]

        def compute(slot, nr):
            bv, bi = bufsets[slot]

            @plsc.parallel_loop(0, nr, unroll=2)
            def _rows(i):
                for t in range(LW // LANES):
                    sl = pl.ds(t * LANES, LANES)
                    plsc.addupdate_scatter(hist, [bi[i, sl]], bv[i, sl])

        items = [(b * BR, BR) for b in range(nblk_full)]
        if part > 0:
            items.append((nblk_full * BR, part))

        base0 = wid * cr
        cps = issue(0, base0 + items[0][0], items[0][1])

        @plsc.parallel_loop(0, BINS_PAD, step=8 * LANES, unroll=2)
        def _zero(i):
            for t in range(8):
                hist[pl.ds(i + t * LANES, LANES)] = jnp.zeros((LANES,), jnp.float32)
        for j in range(len(items)):
            slot = j % 2
            nxt = j + 1
            if nxt < len(items):
                pre = issue(nxt % 2, base0 + items[nxt][0], items[nxt][1])
            for c in cps:
                c.wait()
            compute(slot, items[j][1])
            if nxt < len(items):
                cps = pre

        if tail > 0:
            @pl.when(wid == NW - 1)
            def _tail():
                tcp = issue(0, NW * cr, tail)
                for c in tcp:
                    c.wait()
                compute(0, tail)

        for r in range(R_ROWS):
            pltpu.sync_copy(hist.at[pl.ds(r * R_COLS, R_COLS)],
                            out_hbm.at[wid, r])

    return sc_hist


def _post_body(x_ref, o_ref, *, scale):
    h = jnp.sum(x_ref[...], axis=0)
    m = h * scale

    lane = lax.broadcasted_iota(jnp.int32, (R_ROWS, R_COLS), 1)
    row = lax.broadcasted_iota(jnp.int32, (R_ROWS, R_COLS), 0)

    c = m
    k = 1
    while k < R_COLS:
        shifted = pltpu.roll(c, k, 1)
        c = c + jnp.where(lane >= k, shifted, 0.0)
        k *= 2

    rs = c[:, R_COLS - 1:R_COLS]
    rowi = lax.broadcasted_iota(jnp.int32, (R_ROWS, 1), 0)
    rc = rs
    for k in (1, 2, 4):
        shifted = pltpu.roll(rc, k, 0)
        rc = rc + jnp.where(rowi >= k, shifted, 0.0)

    prefix = c + (rc - rs)

    flat = lane + R_COLS * row
    valid = flat < RES
    pmin = jnp.min(jnp.where(valid, prefix, jnp.inf))
    best = jnp.minimum(pmin, 0.0)

    is_best = jnp.logical_and(valid, prefix == best)
    cnt = jnp.sum(is_best.astype(jnp.float32)) + jnp.where(
        best == 0.0, jnp.float32(1.0), jnp.float32(0.0))
    sidx = jnp.sum(jnp.where(is_best, (flat + 1).astype(jnp.float32), 0.0))
    o_ref[0, 0] = sidx / cnt / 100000.0


def _tc_post(x, scale, interpret=False):
    return pl.pallas_call(
        functools.partial(_post_body, scale=scale),
        out_shape=jax.ShapeDtypeStruct((1, 1), jnp.float32),
        out_specs=pl.BlockSpec(memory_space=pltpu.SMEM),
        interpret=interpret,
    )(x)


def kernel(epes_stat_flow, epes_dyn_flow, moving_mask, dynamicness_scores,
           summaries=0, training=True):
    n = epes_stat_flow.shape[0]
    total = NUM_MOVING + NUM_STILL
    avg_points_per_sample = total / NUM_TRAIN_SAMPLES
    update_weight = 1.0 / min(2.0 * total, 5000.0 * avg_points_per_sample)
    cur_update_weight = (1.0 - update_weight) ** float(n)
    scale = 1.0 - cur_update_weight

    nrows = n // LW
    stat2 = epes_stat_flow.reshape(nrows, LW)
    dyn2 = epes_dyn_flow.reshape(nrows, LW)
    mask2 = moving_mask.reshape(nrows, LW)
    score2 = dynamicness_scores.reshape(nrows, LW)

    impr, idx = _tc_pre(stat2, dyn2, mask2, score2)
    x = _sc_hist_fn(PAD_ROWS)(impr, idx)
    out = _tc_post(x, scale)
    return out.reshape((1,))


# speedup vs baseline: 2.4927x; 1.1622x over previous
import dataclasses
import functools

import jax
import jax.numpy as jnp
from jax import lax
from jax.experimental import pallas as pl
from jax.experimental.pallas import tpu as pltpu
from jax.experimental.pallas import tpu_sc as plsc

RES = 100000
NUM_MOVING = 1500000
NUM_STILL = 500000
NUM_TRAIN_SAMPLES = 10000

R_ROWS = 8
R_COLS = 12544
BINS_PAD = R_ROWS * R_COLS

NC = 2
NS = 16
NW = NC * NS
LANES = 16

BLK = 3584

W_MOV = 1.0 / float(NUM_MOVING)
W_STL = 1.0 / float(NUM_STILL)


def _sc_hist_fn(n):
    c0 = (n // (NW * LANES)) * LANES
    tail = n - NW * c0
    nblk_full = c0 // BLK
    part = c0 - nblk_full * BLK

    mesh = plsc.VectorSubcoreMesh(core_axis_name="c", subcore_axis_name="s")

    cp = pltpu.CompilerParams()
    if "needs_layout_passes" in pltpu.CompilerParams.__dataclass_fields__:
        cp = dataclasses.replace(cp, needs_layout_passes=False)

    @functools.partial(
        pl.kernel,
        compiler_params=cp,
        out_type=jax.ShapeDtypeStruct((NW, R_ROWS, R_COLS), jnp.float32),
        mesh=mesh,
        scratch_types=(
            [pltpu.VMEM((BINS_PAD,), jnp.float32)]
            + [pltpu.VMEM((BLK,), jnp.float32) for _ in range(8)]
            + [pltpu.SemaphoreType.DMA, pltpu.SemaphoreType.DMA]
        ),
    )
    def sc_hist(stat_hbm, dyn_hbm, maskf_hbm, score_hbm, out_hbm,
                hist, bs0, bd0, bm0, bq0, bs1, bd1, bm1, bq1, sem0, sem1):
        wid = lax.axis_index("c") * NS + lax.axis_index("s")
        sems = (sem0, sem1)
        srcs = (stat_hbm, dyn_hbm, maskf_hbm, score_hbm)
        bufsets = ((bs0, bd0, bm0, bq0), (bs1, bd1, bm1, bq1))

        def issue(slot, base, cnt):
            return [pltpu.async_copy(src.at[pl.ds(base, cnt)],
                                     (buf if cnt == BLK else buf.at[pl.ds(0, cnt)]),
                                     sems[slot])
                    for src, buf in zip(srcs, bufsets[slot])]

        def compute(slot, cnt):
            bs, bd, bm, bq = bufsets[slot]

            @plsc.parallel_loop(0, cnt, step=LANES, unroll=2)
            def _vecs(i):
                sl = pl.ds(i, LANES)
                diff = bs[sl] - bd[sl]
                w = jnp.where(bm[sl] > 0.5, jnp.float32(W_MOV), jnp.float32(W_STL))
                impr = diff * w
                idx = (bq[sl] * 100000.0).astype(jnp.int32)
                idx = jnp.minimum(idx, RES - 1)
                plsc.addupdate_scatter(hist, [idx], impr)

        items = [(b * BLK, BLK) for b in range(nblk_full)]
        if part > 0:
            items.append((nblk_full * BLK, part))

        base0 = wid * c0
        cps = issue(0, base0 + items[0][0], items[0][1])

        @plsc.parallel_loop(0, BINS_PAD, step=8 * LANES, unroll=2)
        def _zero(i):
            for t in range(8):
                hist[pl.ds(i + t * LANES, LANES)] = jnp.zeros((LANES,), jnp.float32)

        for j in range(len(items)):
            slot = j % 2
            nxt = j + 1
            if nxt < len(items):
                pre = issue(nxt % 2, base0 + items[nxt][0], items[nxt][1])
            for c in cps:
                c.wait()
            compute(slot, items[j][1])
            if nxt < len(items):
                cps = pre

        if tail > 0:
            @pl.when(wid == NW - 1)
            def _tail():
                tcp = issue(0, NW * c0, tail)
                for c in tcp:
                    c.wait()
                compute(0, tail)

        for r in range(R_ROWS):
            pltpu.sync_copy(hist.at[pl.ds(r * R_COLS, R_COLS)],
                            out_hbm.at[wid, r])

    return sc_hist


def _post_body(x_ref, o_ref, *, scale):
    h = jnp.sum(x_ref[...], axis=0)
    m = h * scale

    lane = lax.broadcasted_iota(jnp.int32, (R_ROWS, R_COLS), 1)
    row = lax.broadcasted_iota(jnp.int32, (R_ROWS, R_COLS), 0)

    c = m
    k = 1
    while k < R_COLS:
        shifted = pltpu.roll(c, k, 1)
        c = c + jnp.where(lane >= k, shifted, 0.0)
        k *= 2

    rs = c[:, R_COLS - 1:R_COLS]
    rowi = lax.broadcasted_iota(jnp.int32, (R_ROWS, 1), 0)
    rc = rs
    for k in (1, 2, 4):
        shifted = pltpu.roll(rc, k, 0)
        rc = rc + jnp.where(rowi >= k, shifted, 0.0)

    prefix = c + (rc - rs)

    flat = lane + R_COLS * row
    valid = flat < RES
    pmin = jnp.min(jnp.where(valid, prefix, jnp.inf))
    best = jnp.minimum(pmin, 0.0)

    is_best = jnp.logical_and(valid, prefix == best)
    cnt = jnp.sum(is_best.astype(jnp.float32)) + jnp.where(
        best == 0.0, jnp.float32(1.0), jnp.float32(0.0))
    sidx = jnp.sum(jnp.where(is_best, (flat + 1).astype(jnp.float32), 0.0))
    o_ref[0, 0] = sidx / cnt / 100000.0


def _tc_post(x, scale, interpret=False):
    return pl.pallas_call(
        functools.partial(_post_body, scale=scale),
        out_shape=jax.ShapeDtypeStruct((1, 1), jnp.float32),
        out_specs=pl.BlockSpec(memory_space=pltpu.SMEM),
        interpret=interpret,
    )(x)


def kernel(epes_stat_flow, epes_dyn_flow, moving_mask, dynamicness_scores,
           summaries=0, training=True):
    n = epes_stat_flow.shape[0]
    total = NUM_MOVING + NUM_STILL
    avg_points_per_sample = total / NUM_TRAIN_SAMPLES
    update_weight = 1.0 / min(2.0 * total, 5000.0 * avg_points_per_sample)
    cur_update_weight = (1.0 - update_weight) ** float(n)
    scale = 1.0 - cur_update_weight

    maskf = moving_mask.astype(jnp.float32)
    x = _sc_hist_fn(n)(epes_stat_flow, epes_dyn_flow, maskf,
                       dynamicness_scores)
    out = _tc_post(x, scale)
    return out.reshape((1,))


# speedup vs baseline: 2.4976x; 1.0020x over previous
import dataclasses
import functools

import jax
import jax.numpy as jnp
from jax import lax
from jax.experimental import pallas as pl
from jax.experimental.pallas import tpu as pltpu
from jax.experimental.pallas import tpu_sc as plsc

RES = 100000
NUM_MOVING = 1500000
NUM_STILL = 500000
NUM_TRAIN_SAMPLES = 10000

R_ROWS = 8
R_COLS = 12544
BINS_PAD = R_ROWS * R_COLS

NC = 2
NS = 16
NW = NC * NS
LANES = 16

BLK = 3584

W_MOV = 1.0 / float(NUM_MOVING)
W_STL = 1.0 / float(NUM_STILL)


def _sc_hist_fn(n):
    c0 = (n // (NW * LANES)) * LANES
    tail = n - NW * c0
    nblk_full = c0 // BLK
    part = c0 - nblk_full * BLK

    mesh = plsc.VectorSubcoreMesh(core_axis_name="c", subcore_axis_name="s")

    cp = pltpu.CompilerParams()
    if "needs_layout_passes" in pltpu.CompilerParams.__dataclass_fields__:
        cp = dataclasses.replace(cp, needs_layout_passes=False)

    @functools.partial(
        pl.kernel,
        compiler_params=cp,
        out_type=jax.ShapeDtypeStruct((NW, R_ROWS, R_COLS), jnp.float32),
        mesh=mesh,
        scratch_types=(
            [pltpu.VMEM((BINS_PAD,), jnp.float32)]
            + [pltpu.VMEM((BLK,), jnp.float32) for _ in range(8)]
            + [pltpu.SemaphoreType.DMA, pltpu.SemaphoreType.DMA]
        ),
    )
    def sc_hist(stat_hbm, dyn_hbm, maskf_hbm, score_hbm, out_hbm,
                hist, bs0, bd0, bm0, bq0, bs1, bd1, bm1, bq1, sem0, sem1):
        wid = lax.axis_index("c") * NS + lax.axis_index("s")
        sems = (sem0, sem1)
        srcs = (stat_hbm, dyn_hbm, maskf_hbm, score_hbm)
        bufsets = ((bs0, bd0, bm0, bq0), (bs1, bd1, bm1, bq1))

        def issue(slot, base, cnt):
            return [pltpu.async_copy(src.at[pl.ds(base, cnt)],
                                     (buf if cnt == BLK else buf.at[pl.ds(0, cnt)]),
                                     sems[slot])
                    for src, buf in zip(srcs, bufsets[slot])]

        def compute(slot, cnt):
            bs, bd, bm, bq = bufsets[slot]

            @plsc.parallel_loop(0, cnt, step=LANES, unroll=2)
            def _vecs(i):
                sl = pl.ds(i, LANES)
                diff = bs[sl] - bd[sl]
                w = jnp.where(bm[sl] > 0.5, jnp.float32(W_MOV), jnp.float32(W_STL))
                impr = diff * w
                idx = (bq[sl] * 100000.0).astype(jnp.int32)
                idx = jnp.minimum(idx, RES - 1)
                plsc.addupdate_scatter(hist, [idx], impr)

        items = [(b * BLK, BLK) for b in range(nblk_full)]
        if part > 0:
            items.append((nblk_full * BLK, part))

        base0 = wid * c0
        cps = issue(0, base0 + items[0][0], items[0][1])

        @plsc.parallel_loop(0, BINS_PAD, step=8 * LANES, unroll=2)
        def _zero(i):
            for t in range(8):
                hist[pl.ds(i + t * LANES, LANES)] = jnp.zeros((LANES,), jnp.float32)

        for j in range(len(items)):
            slot = j % 2
            nxt = j + 1
            if nxt < len(items):
                pre = issue(nxt % 2, base0 + items[nxt][0], items[nxt][1])
            for c in cps:
                c.wait()
            compute(slot, items[j][1])
            if nxt < len(items):
                cps = pre

        if tail > 0:
            @pl.when(wid == NW - 1)
            def _tail():
                tcp = issue(0, NW * c0, tail)
                for c in tcp:
                    c.wait()
                compute(0, tail)

        dumps = [pltpu.async_copy(hist.at[pl.ds(r * R_COLS, R_COLS)],
                                  out_hbm.at[wid, r], sem0)
                 for r in range(R_ROWS)]
        for d in dumps:
            d.wait()

    return sc_hist


def _post_body(x_ref, o_ref, *, scale):
    h = jnp.sum(x_ref[...], axis=0)
    m = h * scale

    lane = lax.broadcasted_iota(jnp.int32, (R_ROWS, R_COLS), 1)
    row = lax.broadcasted_iota(jnp.int32, (R_ROWS, R_COLS), 0)

    c = m
    k = 1
    while k < R_COLS:
        shifted = pltpu.roll(c, k, 1)
        c = c + jnp.where(lane >= k, shifted, 0.0)
        k *= 2

    rs = c[:, R_COLS - 1:R_COLS]
    rowi = lax.broadcasted_iota(jnp.int32, (R_ROWS, 1), 0)
    rc = rs
    for k in (1, 2, 4):
        shifted = pltpu.roll(rc, k, 0)
        rc = rc + jnp.where(rowi >= k, shifted, 0.0)

    prefix = c + (rc - rs)

    flat = lane + R_COLS * row
    valid = flat < RES
    pmin = jnp.min(jnp.where(valid, prefix, jnp.inf))
    best = jnp.minimum(pmin, 0.0)

    is_best = jnp.logical_and(valid, prefix == best)
    cnt = jnp.sum(is_best.astype(jnp.float32)) + jnp.where(
        best == 0.0, jnp.float32(1.0), jnp.float32(0.0))
    sidx = jnp.sum(jnp.where(is_best, (flat + 1).astype(jnp.float32), 0.0))
    o_ref[0, 0] = sidx / cnt / 100000.0


def _tc_post(x, scale, interpret=False):
    return pl.pallas_call(
        functools.partial(_post_body, scale=scale),
        out_shape=jax.ShapeDtypeStruct((1, 1), jnp.float32),
        out_specs=pl.BlockSpec(memory_space=pltpu.SMEM),
        interpret=interpret,
    )(x)


def kernel(epes_stat_flow, epes_dyn_flow, moving_mask, dynamicness_scores,
           summaries=0, training=True):
    n = epes_stat_flow.shape[0]
    total = NUM_MOVING + NUM_STILL
    avg_points_per_sample = total / NUM_TRAIN_SAMPLES
    update_weight = 1.0 / min(2.0 * total, 5000.0 * avg_points_per_sample)
    cur_update_weight = (1.0 - update_weight) ** float(n)
    scale = 1.0 - cur_update_weight

    maskf = moving_mask.astype(jnp.float32)
    x = _sc_hist_fn(n)(epes_stat_flow, epes_dyn_flow, maskf,
                       dynamicness_scores)
    out = _tc_post(x, scale)
    return out.reshape((1,))
